# Initial kernel scaffold; baseline (speedup 1.0000x reference)
import jax
import jax.numpy as jnp
from jax.experimental import pallas as pl


def kernel(x, edge_index, batch, W1, b1, W2, b2, W3, b3, Wl, bl):
    raise NotImplementedError("write your pallas kernel here")


# speedup vs baseline: 11.0821x; 11.0821x over previous
import functools

import jax
import jax.numpy as jnp
from jax import lax
from jax.experimental import pallas as pl
from jax.experimental.pallas import tpu as pltpu
from jax.experimental.pallas import tpu_sc as plsc

N_NODES = 10000
N_EDGES = 320000
D = 128
N_GRAPHS = 64
D_OUT = 10

NC = 2
NS = 16
NW = NC * NS

N_PAD = 10240
ROWS_PER_TILE = N_PAD // NS
E_PER_TILE = N_EDGES // NW
CHUNK = 80
N_CHUNKS = E_PER_TILE // CHUNK

BLK = 640
GRID = N_PAD // BLK

_sc_mesh = plsc.VectorSubcoreMesh(
    core_axis_name="c", subcore_axis_name="s", num_cores=NC, num_subcores=NS)


def _deg_body(dst_hbm, zeros_hbm, out_hbm, didx_v, ones_v, deg_sh, sem):
    c = lax.axis_index("c")
    s = lax.axis_index("s")
    wid = s * NC + c
    base = wid * E_PER_TILE

    for k in range(CHUNK // 16):
        ones_v[pl.ds(k * 16, 16)] = jnp.ones((16,), jnp.float32)

    pltpu.sync_copy(zeros_hbm, deg_sh.at[pl.ds(s * ROWS_PER_TILE, ROWS_PER_TILE)])
    plsc.subcore_barrier()

    def body(i, carry):
        off = base + i * CHUNK
        pltpu.sync_copy(dst_hbm.at[pl.ds(off, CHUNK)], didx_v)
        pltpu.sync_copy(ones_v, deg_sh.at[didx_v], add=True)
        return carry

    lax.fori_loop(0, N_CHUNKS, body, 0)
    plsc.subcore_barrier()
    pltpu.sync_copy(deg_sh.at[pl.ds(s * ROWS_PER_TILE, ROWS_PER_TILE)],
                    out_hbm.at[c, pl.ds(s * ROWS_PER_TILE, ROWS_PER_TILE)])


_deg = pl.kernel(
    _deg_body,
    out_type=jax.ShapeDtypeStruct((NC, N_PAD), jnp.float32),
    mesh=_sc_mesh,
    scratch_types=[
        pltpu.VMEM((CHUNK,), jnp.int32),
        pltpu.VMEM((CHUNK,), jnp.float32),
        pltpu.VMEM_SHARED((N_PAD,), jnp.float32),
        pltpu.SemaphoreType.DMA,
    ],
)


def _spmm_body(h_hbm, src_hbm, dst_hbm, zeros_hbm, out_hbm,
               sidx_v, didx_v, rows_v, acc_sh, sem):
    c = lax.axis_index("c")
    s = lax.axis_index("s")
    wid = s * NC + c
    base = wid * E_PER_TILE

    pltpu.sync_copy(zeros_hbm, acc_sh.at[pl.ds(s * ROWS_PER_TILE, ROWS_PER_TILE)])
    plsc.subcore_barrier()

    def body(i, carry):
        off = base + i * CHUNK
        pltpu.sync_copy(src_hbm.at[pl.ds(off, CHUNK)], sidx_v)
        pltpu.sync_copy(dst_hbm.at[pl.ds(off, CHUNK)], didx_v)
        pltpu.async_copy(h_hbm.at[sidx_v], rows_v, sem).wait()
        pltpu.sync_copy(rows_v, acc_sh.at[didx_v], add=True)
        return carry

    lax.fori_loop(0, N_CHUNKS, body, 0)
    plsc.subcore_barrier()
    pltpu.sync_copy(acc_sh.at[pl.ds(s * ROWS_PER_TILE, ROWS_PER_TILE)],
                    out_hbm.at[c, pl.ds(s * ROWS_PER_TILE, ROWS_PER_TILE)])


_spmm = pl.kernel(
    _spmm_body,
    out_type=jax.ShapeDtypeStruct((NC, N_PAD, D), jnp.float32),
    mesh=_sc_mesh,
    scratch_types=[
        pltpu.VMEM((CHUNK,), jnp.int32),
        pltpu.VMEM((CHUNK,), jnp.int32),
        pltpu.VMEM((CHUNK, D), jnp.float32),
        pltpu.VMEM_SHARED((N_PAD, D), jnp.float32),
        pltpu.SemaphoreType.DMA,
    ],
)


def _first_body(x_ref, w_ref, deg_ref, hp_ref, dinv_ref):
    deg = deg_ref[0] + deg_ref[1] + 1.0
    dinv = lax.rsqrt(deg)
    h = jnp.dot(x_ref[...], w_ref[...], preferred_element_type=jnp.float32)
    hp_ref[...] = h * dinv
    dinv_ref[...] = dinv


_k_first = pl.pallas_call(
    _first_body,
    grid=(GRID,),
    in_specs=[
        pl.BlockSpec((BLK, D), lambda i: (i, 0)),
        pl.BlockSpec((D, D), lambda i: (0, 0)),
        pl.BlockSpec((NC, BLK, 1), lambda i: (0, i, 0)),
    ],
    out_specs=[
        pl.BlockSpec((BLK, D), lambda i: (i, 0)),
        pl.BlockSpec((BLK, 1), lambda i: (i, 0)),
    ],
    out_shape=[
        jax.ShapeDtypeStruct((N_PAD, D), jnp.float32),
        jax.ShapeDtypeStruct((N_PAD, 1), jnp.float32),
    ],
)


def _mid_body(acc_ref, hp_ref, dinv_ref, b_ref, w_ref, out_ref):
    dinv = dinv_ref[...]
    tot = acc_ref[0] + acc_ref[1] + hp_ref[...]
    h = jnp.maximum(dinv * tot + b_ref[...], 0.0)
    out_ref[...] = jnp.dot(h, w_ref[...], preferred_element_type=jnp.float32) * dinv


_k_mid = pl.pallas_call(
    _mid_body,
    grid=(GRID,),
    in_specs=[
        pl.BlockSpec((NC, BLK, D), lambda i: (0, i, 0)),
        pl.BlockSpec((BLK, D), lambda i: (i, 0)),
        pl.BlockSpec((BLK, 1), lambda i: (i, 0)),
        pl.BlockSpec((1, D), lambda i: (0, 0)),
        pl.BlockSpec((D, D), lambda i: (0, 0)),
    ],
    out_specs=pl.BlockSpec((BLK, D), lambda i: (i, 0)),
    out_shape=jax.ShapeDtypeStruct((N_PAD, D), jnp.float32),
)


def _final_body(acc_ref, hp_ref, dinv_ref, b_ref, batch_ref, wl_ref, bl_ref,
                out_ref, pools, counts):
    i = pl.program_id(0)

    @pl.when(i == 0)
    def _():
        pools[...] = jnp.zeros_like(pools)
        counts[...] = jnp.zeros_like(counts)

    dinv = dinv_ref[...]
    h3 = dinv * (acc_ref[0] + acc_ref[1] + hp_ref[...]) + b_ref[...]
    bb = batch_ref[0]
    gids = lax.broadcasted_iota(jnp.int32, (BLK, N_GRAPHS), 1)
    oh = (bb == gids).astype(jnp.float32)
    dn = (((0,), (0,)), ((), ()))
    pools[...] += lax.dot_general(oh, h3, dn, preferred_element_type=jnp.float32)
    ones = jnp.ones((BLK, 1), jnp.float32)
    counts[...] += lax.dot_general(oh, ones, dn, preferred_element_type=jnp.float32)

    @pl.when(i == GRID - 1)
    def _():
        g = pools[...] / jnp.maximum(counts[...], 1.0)
        logits = jnp.dot(g, wl_ref[...], preferred_element_type=jnp.float32) + bl_ref[...]
        m = jnp.max(logits, axis=1, keepdims=True)
        z = logits - m
        lse = jnp.log(jnp.sum(jnp.exp(z), axis=1, keepdims=True))
        out_ref[...] = z - lse


_k_final = pl.pallas_call(
    _final_body,
    grid=(GRID,),
    in_specs=[
        pl.BlockSpec((NC, BLK, D), lambda i: (0, i, 0)),
        pl.BlockSpec((BLK, D), lambda i: (i, 0)),
        pl.BlockSpec((BLK, 1), lambda i: (i, 0)),
        pl.BlockSpec((1, D), lambda i: (0, 0)),
        pl.BlockSpec((1, BLK, 1), lambda i: (i, 0, 0)),
        pl.BlockSpec((D, D_OUT), lambda i: (0, 0)),
        pl.BlockSpec((1, D_OUT), lambda i: (0, 0)),
    ],
    out_specs=pl.BlockSpec((N_GRAPHS, D_OUT), lambda i: (0, 0)),
    out_shape=jax.ShapeDtypeStruct((N_GRAPHS, D_OUT), jnp.float32),
    scratch_shapes=[
        pltpu.VMEM((N_GRAPHS, D), jnp.float32),
        pltpu.VMEM((N_GRAPHS, 1), jnp.float32),
    ],
)


@jax.jit
def kernel(x, edge_index, batch, W1, b1, W2, b2, W3, b3, Wl, bl):
    src = edge_index[0].astype(jnp.int32)
    dst = edge_index[1].astype(jnp.int32)

    x_pad = jnp.pad(x.astype(jnp.float32), ((0, N_PAD - N_NODES), (0, 0)))
    batch_pad = jnp.concatenate(
        [batch.astype(jnp.int32),
         jnp.full((N_PAD - N_NODES,), -1, jnp.int32)]).reshape(GRID, BLK, 1)

    zeros_row = jnp.zeros((ROWS_PER_TILE,), jnp.float32)
    zeros_nd = jnp.zeros((ROWS_PER_TILE, D), jnp.float32)

    deg = _deg(dst, zeros_row)
    deg3 = deg.reshape(NC, N_PAD, 1)

    hp1, dinv = _k_first(x_pad, W1, deg3)
    acc1 = _spmm(hp1, src, dst, zeros_nd)
    hp2 = _k_mid(acc1, hp1, dinv, b1.reshape(1, D), W2)
    acc2 = _spmm(hp2, src, dst, zeros_nd)
    hp3 = _k_mid(acc2, hp2, dinv, b2.reshape(1, D), W3)
    acc3 = _spmm(hp3, src, dst, zeros_nd)
    out = _k_final(acc3, hp3, dinv, b3.reshape(1, D), batch_pad,
                   Wl, bl.reshape(1, D_OUT))
    return out
